# Initial kernel scaffold; baseline (speedup 1.0000x reference)
import jax
import jax.numpy as jnp
from jax.experimental import pallas as pl


def kernel(hidden_states, gate_w, e_score_correction_bias, w13, w2, shared_w13, shared_w2):
    raise NotImplementedError("write your pallas kernel here")


# speedup vs baseline: 1.3955x; 1.3955x over previous
import numpy as np

import jax
import jax.numpy as jnp
from jax.experimental import pallas as pl
from jax.experimental.pallas import tpu as pltpu

RSF = 2.5
NG = 4
TG = 2
TOPK = 2


def _rank_lt(vals, k):
    Tn, L = vals.shape
    lane = jax.lax.broadcasted_iota(jnp.int32, (Tn, L), 1)
    cols = []
    for j in range(L):
        col = vals[:, j : j + 1]
        gt = (vals > col).astype(jnp.float32)
        eq_lo = jnp.logical_and(vals == col, lane < j).astype(jnp.float32)
        cols.append(jnp.sum(gt + eq_lo, axis=1, keepdims=True))
    rank = jnp.concatenate(cols, axis=1)
    return (rank < float(k)).astype(jnp.float32)


def _moe_kernel(x_ref, gw_ref, bias_ref, w13_ref, w2_ref, o_ref, cw_ref):
    e = pl.program_id(0)

    @pl.when(e == 0)
    def _routing():
        x = x_ref[...]
        E = gw_ref.shape[0]
        per = E // NG
        logits = jax.lax.dot_general(
            x, gw_ref[...], (((1,), (1,)), ((), ())),
            preferred_element_type=jnp.float32)
        scores = jax.nn.sigmoid(logits)
        sfc = scores + bias_ref[...]
        lane_e = jax.lax.broadcasted_iota(jnp.int32, (x.shape[0], E), 1)
        grp_of_e = lane_e // per
        gs = jnp.concatenate(
            [jnp.sum(jnp.where(grp_of_e == g, sfc, 0.0), axis=1,
                     keepdims=True) for g in range(NG)], axis=1)
        gmask = _rank_lt(gs, TG)
        emask = jnp.concatenate(
            [jnp.broadcast_to(gmask[:, g : g + 1], (x.shape[0], per))
             for g in range(NG)], axis=1)
        masked = jnp.where(emask > 0.0, sfc, -jnp.inf)
        chosen = _rank_lt(masked, TOPK)
        w = scores * chosen
        w = w / (jnp.sum(w, axis=1, keepdims=True) + 1e-20)
        cw_ref[:, :E] = w * RSF
        cw_ref[:, E:] = jnp.ones((x.shape[0], cw_ref.shape[1] - E),
                                 jnp.float32)

    x = x_ref[...]
    gu = jnp.dot(x, w13_ref[0], preferred_element_type=jnp.float32)
    dff = gu.shape[1] // 2
    g = gu[:, :dff]
    u = gu[:, dff:]
    h = (g * jax.nn.sigmoid(g)) * u
    contrib = jnp.dot(h, w2_ref[0], preferred_element_type=jnp.float32)
    lane = jax.lax.broadcasted_iota(jnp.int32, cw_ref.shape, 1)
    wcol = jnp.sum(jnp.where(lane == e, cw_ref[...], 0.0), axis=1,
                   keepdims=True)
    contrib = contrib * wcol

    @pl.when(e == 0)
    def _init():
        o_ref[...] = contrib

    @pl.when(e != 0)
    def _acc():
        o_ref[...] += contrib


def kernel(hidden_states, gate_w, e_score_correction_bias, w13, w2,
           shared_w13, shared_w2):
    T, D = hidden_states.shape
    E, _, DFF2 = w13.shape
    DFF = DFF2 // 2
    SH = shared_w13.shape[1] // 2
    NSH = SH // DFF

    sg = shared_w13[:, :SH].reshape(D, NSH, DFF)
    su = shared_w13[:, SH:].reshape(D, NSH, DFF)
    sh13 = jnp.concatenate([sg, su], axis=-1).transpose(1, 0, 2)
    sh2 = shared_w2.reshape(NSH, DFF, D)
    w13_all = jnp.concatenate([w13, sh13], axis=0)
    w2_all = jnp.concatenate([w2, sh2], axis=0)
    NE = E + NSH

    bias2d = e_score_correction_bias.reshape(1, E)

    out = pl.pallas_call(
        _moe_kernel,
        grid=(NE,),
        in_specs=[
            pl.BlockSpec((T, D), lambda e: (0, 0)),
            pl.BlockSpec((E, D), lambda e: (0, 0)),
            pl.BlockSpec((1, E), lambda e: (0, 0)),
            pl.BlockSpec((1, D, DFF2), lambda e: (e, 0, 0)),
            pl.BlockSpec((1, DFF, D), lambda e: (e, 0, 0)),
        ],
        out_specs=pl.BlockSpec((T, D), lambda e: (0, 0)),
        out_shape=jax.ShapeDtypeStruct((T, D), hidden_states.dtype),
        scratch_shapes=[pltpu.VMEM((T, NE), jnp.float32)],
    )(hidden_states, gate_w, bias2d, w13_all, w2_all)
    return out


# speedup vs baseline: 1.6039x; 1.1493x over previous
import numpy as np

import jax
import jax.numpy as jnp
from jax.experimental import pallas as pl
from jax.experimental.pallas import tpu as pltpu

RSF = 2.5
NG = 4
TG = 2
TOPK = 2


def _rank_lt(vals, k):
    Tn, L = vals.shape
    lane = jax.lax.broadcasted_iota(jnp.int32, (Tn, L), 1)
    cols = []
    for j in range(L):
        col = vals[:, j : j + 1]
        gt = (vals > col).astype(jnp.float32)
        eq_lo = jnp.logical_and(vals == col, lane < j).astype(jnp.float32)
        cols.append(jnp.sum(gt + eq_lo, axis=1, keepdims=True))
    rank = jnp.concatenate(cols, axis=1)
    return (rank < float(k)).astype(jnp.float32)


def _moe_kernel(x_ref, gw_ref, bias_ref, w13_ref, w2_ref, o_ref, cw_ref,
                xb_ref):
    e = pl.program_id(0)

    @pl.when(e == 0)
    def _routing():
        x = x_ref[...]
        xb_ref[...] = x.astype(jnp.bfloat16)
        E = gw_ref.shape[0]
        per = E // NG
        logits = jax.lax.dot_general(
            x, gw_ref[...], (((1,), (1,)), ((), ())),
            preferred_element_type=jnp.float32)
        scores = jax.nn.sigmoid(logits)
        sfc = scores + bias_ref[...]
        lane_e = jax.lax.broadcasted_iota(jnp.int32, (x.shape[0], E), 1)
        grp_of_e = lane_e // per
        gs = jnp.concatenate(
            [jnp.sum(jnp.where(grp_of_e == g, sfc, 0.0), axis=1,
                     keepdims=True) for g in range(NG)], axis=1)
        gmask = _rank_lt(gs, TG)
        emask = jnp.concatenate(
            [jnp.broadcast_to(gmask[:, g : g + 1], (x.shape[0], per))
             for g in range(NG)], axis=1)
        masked = jnp.where(emask > 0.0, sfc, -jnp.inf)
        chosen = _rank_lt(masked, TOPK)
        w = scores * chosen
        w = w / (jnp.sum(w, axis=1, keepdims=True) + 1e-20)
        cw_ref[:, :E] = w * RSF
        cw_ref[:, E:] = jnp.ones((x.shape[0], cw_ref.shape[1] - E),
                                 jnp.float32)

    gu = jnp.dot(xb_ref[...], w13_ref[0], preferred_element_type=jnp.float32)
    dff = gu.shape[1] // 2
    g = gu[:, :dff]
    u = gu[:, dff:]
    h = (g * jax.nn.sigmoid(g)) * u
    contrib = jnp.dot(h.astype(jnp.bfloat16), w2_ref[0],
                      preferred_element_type=jnp.float32)
    lane = jax.lax.broadcasted_iota(jnp.int32, cw_ref.shape, 1)
    wcol = jnp.sum(jnp.where(lane == e, cw_ref[...], 0.0), axis=1,
                   keepdims=True)
    contrib = contrib * wcol

    @pl.when(e == 0)
    def _init():
        o_ref[...] = contrib

    @pl.when(e != 0)
    def _acc():
        o_ref[...] += contrib


def kernel(hidden_states, gate_w, e_score_correction_bias, w13, w2,
           shared_w13, shared_w2):
    T, D = hidden_states.shape
    E, _, DFF2 = w13.shape
    DFF = DFF2 // 2
    SH = shared_w13.shape[1] // 2
    NSH = SH // DFF

    sg = shared_w13[:, :SH].reshape(D, NSH, DFF)
    su = shared_w13[:, SH:].reshape(D, NSH, DFF)
    sh13 = jnp.concatenate([sg, su], axis=-1).transpose(1, 0, 2)
    sh2 = shared_w2.reshape(NSH, DFF, D)
    w13_all = jnp.concatenate([w13, sh13], axis=0)
    w2_all = jnp.concatenate([w2, sh2], axis=0)
    NE = E + NSH

    bias2d = e_score_correction_bias.reshape(1, E)

    out = pl.pallas_call(
        _moe_kernel,
        grid=(NE,),
        in_specs=[
            pl.BlockSpec((T, D), lambda e: (0, 0)),
            pl.BlockSpec((E, D), lambda e: (0, 0)),
            pl.BlockSpec((1, E), lambda e: (0, 0)),
            pl.BlockSpec((1, D, DFF2), lambda e: (e, 0, 0)),
            pl.BlockSpec((1, DFF, D), lambda e: (e, 0, 0)),
        ],
        out_specs=pl.BlockSpec((T, D), lambda e: (0, 0)),
        out_shape=jax.ShapeDtypeStruct((T, D), hidden_states.dtype),
        scratch_shapes=[pltpu.VMEM((T, NE), jnp.float32),
                        pltpu.VMEM((T, D), jnp.bfloat16)],
    )(hidden_states, gate_w, bias2d, w13_all.astype(jnp.bfloat16),
      w2_all.astype(jnp.bfloat16))
    return out
